# baseline (speedup 1.0000x reference)
import functools

import jax
import jax.numpy as jnp
from jax import lax
from jax.experimental import pallas as pl
from jax.experimental.pallas import tpu as pltpu
from jax.experimental.pallas import tpu_sc as plsc

_W = 256
_SLAB = 128


@functools.cache
def _make_repack(V, D):
    info = plsc.get_sparse_core_info()
    NC, NS, L = info.num_cores, info.num_subcores, info.num_lanes
    NW = NC * NS
    assert D == 64 and L == 16
    n_slabs = (V + _SLAB - 1) // _SLAB
    has_tail = V % _SLAB != 0
    mesh = plsc.VectorSubcoreMesh(core_axis_name="c", subcore_axis_name="s")

    @functools.partial(
        pl.kernel,
        mesh=mesh,
        out_type=jax.ShapeDtypeStruct((V, 2 * D), jnp.float32),
        scratch_types=[
            pltpu.VMEM((2, D, _SLAB), jnp.float32),
            pltpu.VMEM((2, _SLAB, 2 * D + 1), jnp.float32),
            pltpu.SemaphoreType.DMA,
            pltpu.SemaphoreType.DMA,
            pltpu.SemaphoreType.DMA,
            pltpu.SemaphoreType.DMA,
        ],
        compiler_params=pltpu.CompilerParams(
            use_tc_tiling_on_sc=True, needs_layout_passes=False),
    )
    def repack_kernel(tt_hbm, ttail_hbm, t3_hbm, gi_v, go_v,
                      isem0, isem1, osem0, osem1):
        isem = (isem0, isem1)
        osem = (osem0, osem1)
        wid = lax.axis_index("s") * NC + lax.axis_index("c")
        n_w = n_slabs // NW + jnp.where(wid < n_slabs % NW, 1, 0)

        def slab_of(i):
            return wid + i * NW

        def start_in(i, b):
            s = slab_of(i)
            if has_tail:
                @pl.when(s == n_slabs - 1)
                def _():
                    pltpu.async_copy(ttail_hbm, gi_v.at[b], isem[b])

                @pl.when(s < n_slabs - 1)
                def _():
                    c0 = pl.multiple_of(s * _SLAB, _SLAB)
                    pltpu.async_copy(
                        tt_hbm.at[:, pl.ds(c0, _SLAB)], gi_v.at[b], isem[b])
            else:
                c0 = pl.multiple_of(s * _SLAB, _SLAB)
                pltpu.async_copy(
                    tt_hbm.at[:, pl.ds(c0, _SLAB)], gi_v.at[b], isem[b])

        def wait_in(b):
            pltpu.make_async_copy(
                tt_hbm.at[:, pl.ds(0, _SLAB)], gi_v.at[b], isem[b]).wait()

        def transpose(b):
            gi2 = gi_v.at[b]
            go2 = go_v.at[b]

            def dbody(d, carry):
                dv = jnp.full((L,), 0, jnp.int32) + d
                for k in range(_SLAB // L):
                    v = gi2[d, pl.ds(k * L, L)]
                    cv = lax.iota(jnp.int32, L) + k * L
                    plsc.store_scatter(go2, [cv, dv], v)
                return carry

            lax.fori_loop(0, D, dbody, 0, unroll=4)

        def row0_of(i):
            s = slab_of(i)
            if has_tail:
                return jnp.where(s == n_slabs - 1, V - _SLAB, s * _SLAB)
            return s * _SLAB

        def start_out(i, b):
            r0 = pl.multiple_of(row0_of(i), 64)
            pltpu.async_copy(
                go_v.at[b, slice(None), pl.ds(0, 2 * D)],
                t3_hbm.at[pl.ds(r0, _SLAB)], osem[b])

        def wait_out(b):
            pltpu.make_async_copy(
                go_v.at[b, slice(None), pl.ds(0, 2 * D)],
                t3_hbm.at[pl.ds(0, _SLAB)], osem[b]).wait()

        start_in(0, 0)

        def body(go_i, carry):
            for b in range(2):
                i_var = go_i * 2 + b

                @pl.when(i_var + 1 < n_w)
                def _():
                    start_in(i_var + 1, 1 - b)

                @pl.when(i_var < n_w)
                def _():
                    wait_in(b)

                    @pl.when(i_var >= 2)
                    def _():
                        wait_out(b)

                    transpose(b)
                    start_out(i_var, b)
            return carry

        n_outer = (n_slabs // NW + 2) // 2
        lax.fori_loop(0, n_outer, body, 0)
        wait_out(0)
        wait_out(1)

    return repack_kernel


@functools.cache
def _make_gather(V, D, B, H):
    info = plsc.get_sparse_core_info()
    NC, NS, L = info.num_cores, info.num_subcores, info.num_lanes
    NW = NC * NS
    N = B * H
    assert D == 64 and L == 16 and B % _W == 0
    chunks_per_h = B // _W
    n_chunks = N // _W
    assert n_chunks % NW == 0
    nc_per_w = n_chunks // NW
    assert nc_per_w % 2 == 0
    mesh = plsc.VectorSubcoreMesh(core_axis_name="c", subcore_axis_name="s")

    @functools.partial(
        pl.kernel,
        mesh=mesh,
        out_type=jax.ShapeDtypeStruct((H, D, B), jnp.float32),
        scratch_types=[
            pltpu.VMEM((_W,), jnp.int32),
            pltpu.VMEM((_W,), jnp.int32),
            pltpu.VMEM((2, _W, 2 * D), jnp.float32),
            pltpu.VMEM((2, D, _W + 1), jnp.float32),
            pltpu.SemaphoreType.DMA,
            pltpu.SemaphoreType.DMA,
            pltpu.SemaphoreType.DMA,
            pltpu.SemaphoreType.DMA,
        ],
        compiler_params=pltpu.CompilerParams(
            use_tc_tiling_on_sc=True, needs_layout_passes=False),
    )
    def gather_kernel(idx_hbm, t3_hbm, out_hbm, idx_v0, idx_v1, g_v, o_v,
                      gsem0, gsem1, wsem0, wsem1):
        idx_v = (idx_v0, idx_v1)
        gsem = (gsem0, gsem1)
        wsem = (wsem0, wsem1)
        wid = lax.axis_index("s") * NC + lax.axis_index("c")
        c0 = wid * nc_per_w

        def stage_and_gather(i, b):
            off = pl.multiple_of((c0 + i) * _W, _W)
            pltpu.sync_copy(idx_hbm.at[pl.ds(off, _W)], idx_v[b])
            pltpu.async_copy(t3_hbm.at[idx_v[b]], g_v.at[b], gsem[b])

        def wait_gather(b):
            pltpu.make_async_copy(
                t3_hbm.at[idx_v[b]], g_v.at[b], gsem[b]).wait()

        def shuffle(b):
            g2 = g_v.at[b]
            o2 = o_v.at[b]

            def jbody(j, carry):
                jv = jnp.full((L,), 0, jnp.int32) + j
                for k in range(D // L):
                    v = g2[j, pl.ds(k * L, L)]
                    dv = lax.iota(jnp.int32, L) + k * L
                    plsc.store_scatter(o2, [dv, jv], v)
                return carry

            lax.fori_loop(0, _W, jbody, 0, unroll=4)

        def start_write(i, b):
            c = c0 + i
            h = c // chunks_per_h
            b0 = pl.multiple_of((c % chunks_per_h) * _W, _W)
            pltpu.async_copy(
                o_v.at[b, slice(None), pl.ds(0, _W)],
                out_hbm.at[h, slice(None), pl.ds(b0, _W)], wsem[b])

        def wait_write(b):
            pltpu.make_async_copy(
                o_v.at[b, slice(None), pl.ds(0, _W)],
                out_hbm.at[0, slice(None), pl.ds(0, _W)], wsem[b]).wait()

        stage_and_gather(0, 0)

        def body(go_i, carry):
            for b in range(2):
                i_var = go_i * 2 + b

                @pl.when(i_var + 1 < nc_per_w)
                def _():
                    stage_and_gather(i_var + 1, 1 - b)

                wait_gather(b)

                @pl.when(i_var >= 2)
                def _():
                    wait_write(b)

                shuffle(b)
                start_write(i_var, b)
            return carry

        lax.fori_loop(0, nc_per_w // 2, body, 0)
        wait_write(0)
        wait_write(1)

    return gather_kernel


def kernel(x, table):
    B, H = x.shape
    V, D = table.shape
    idx = x.T.reshape(B * H).astype(jnp.int32)
    tt = table.T
    ttail = lax.slice(tt, (0, V - _SLAB), (D, V))
    t3 = _make_repack(V, D)(tt, ttail)
    out_phys = _make_gather(V, D, B, H)(idx, t3)
    return jnp.transpose(out_phys, (2, 0, 1))


# speedup vs baseline: 1.3894x; 1.3894x over previous
import functools

import jax
import jax.numpy as jnp
from jax import lax
from jax.experimental import pallas as pl
from jax.experimental.pallas import tpu as pltpu
from jax.experimental.pallas import tpu_sc as plsc

_W = 256
_SLAB = 128


@functools.cache
def _make_repack(V, D):
    info = plsc.get_sparse_core_info()
    NC, NS, L = info.num_cores, info.num_subcores, info.num_lanes
    NW = NC * NS
    assert D == 64 and L == 16
    n_slabs = (V + _SLAB - 1) // _SLAB
    has_tail = V % _SLAB != 0
    mesh = plsc.VectorSubcoreMesh(core_axis_name="c", subcore_axis_name="s")

    @functools.partial(
        pl.kernel,
        mesh=mesh,
        out_type=jax.ShapeDtypeStruct((V, 2 * D), jnp.float32),
        scratch_types=[
            pltpu.VMEM((2, D, _SLAB), jnp.float32),
            pltpu.VMEM((2, _SLAB, 2 * D + 1), jnp.float32),
            pltpu.SemaphoreType.DMA,
            pltpu.SemaphoreType.DMA,
            pltpu.SemaphoreType.DMA,
            pltpu.SemaphoreType.DMA,
        ],
        compiler_params=pltpu.CompilerParams(
            use_tc_tiling_on_sc=True, needs_layout_passes=False),
    )
    def repack_kernel(tt_hbm, ttail_hbm, t3_hbm, gi_v, go_v,
                      isem0, isem1, osem0, osem1):
        isem = (isem0, isem1)
        osem = (osem0, osem1)
        wid = lax.axis_index("s") * NC + lax.axis_index("c")
        n_w = n_slabs // NW + jnp.where(wid < n_slabs % NW, 1, 0)

        def slab_of(i):
            return wid + i * NW

        def start_in(i, b):
            s = slab_of(i)
            if has_tail:
                @pl.when(s == n_slabs - 1)
                def _():
                    pltpu.async_copy(ttail_hbm, gi_v.at[b], isem[b])

                @pl.when(s < n_slabs - 1)
                def _():
                    c0 = pl.multiple_of(s * _SLAB, _SLAB)
                    pltpu.async_copy(
                        tt_hbm.at[:, pl.ds(c0, _SLAB)], gi_v.at[b], isem[b])
            else:
                c0 = pl.multiple_of(s * _SLAB, _SLAB)
                pltpu.async_copy(
                    tt_hbm.at[:, pl.ds(c0, _SLAB)], gi_v.at[b], isem[b])

        def wait_in(b):
            pltpu.make_async_copy(
                tt_hbm.at[:, pl.ds(0, _SLAB)], gi_v.at[b], isem[b]).wait()

        def transpose(b):
            gi2 = gi_v.at[b]
            go2 = go_v.at[b]

            def dbody(d):
                dv = jnp.full((L,), 0, jnp.int32) + d
                for k in range(_SLAB // L):
                    v = gi2[d, pl.ds(k * L, L)]
                    cv = lax.iota(jnp.int32, L) + k * L
                    plsc.store_scatter(go2, [cv, dv], v)

            plsc.parallel_loop(0, D, unroll=4)(dbody)

        def row0_of(i):
            s = slab_of(i)
            if has_tail:
                return jnp.where(s == n_slabs - 1, V - _SLAB, s * _SLAB)
            return s * _SLAB

        def start_out(i, b):
            r0 = pl.multiple_of(row0_of(i), 64)
            pltpu.async_copy(
                go_v.at[b, slice(None), pl.ds(0, 2 * D)],
                t3_hbm.at[pl.ds(r0, _SLAB)], osem[b])

        def wait_out(b):
            pltpu.make_async_copy(
                go_v.at[b, slice(None), pl.ds(0, 2 * D)],
                t3_hbm.at[pl.ds(0, _SLAB)], osem[b]).wait()

        start_in(0, 0)

        def body(go_i, carry):
            for b in range(2):
                i_var = go_i * 2 + b

                @pl.when(i_var + 1 < n_w)
                def _():
                    start_in(i_var + 1, 1 - b)

                @pl.when(i_var < n_w)
                def _():
                    wait_in(b)

                    @pl.when(i_var >= 2)
                    def _():
                        wait_out(b)

                    transpose(b)
                    start_out(i_var, b)
            return carry

        n_outer = (n_slabs // NW + 2) // 2
        lax.fori_loop(0, n_outer, body, 0)
        wait_out(0)
        wait_out(1)

    return repack_kernel


@functools.cache
def _make_gather(V, D, B, H):
    info = plsc.get_sparse_core_info()
    NC, NS, L = info.num_cores, info.num_subcores, info.num_lanes
    NW = NC * NS
    N = B * H
    assert D == 64 and L == 16 and B % _W == 0
    chunks_per_h = B // _W
    n_chunks = N // _W
    assert n_chunks % NW == 0
    nc_per_w = n_chunks // NW
    assert nc_per_w % 2 == 0
    mesh = plsc.VectorSubcoreMesh(core_axis_name="c", subcore_axis_name="s")

    @functools.partial(
        pl.kernel,
        mesh=mesh,
        out_type=jax.ShapeDtypeStruct((H, D, B), jnp.float32),
        scratch_types=[
            pltpu.VMEM((_W,), jnp.int32),
            pltpu.VMEM((_W,), jnp.int32),
            pltpu.VMEM((2, _W, 2 * D), jnp.float32),
            pltpu.VMEM((2, D, _W + 1), jnp.float32),
            pltpu.SemaphoreType.DMA,
            pltpu.SemaphoreType.DMA,
            pltpu.SemaphoreType.DMA,
            pltpu.SemaphoreType.DMA,
        ],
        compiler_params=pltpu.CompilerParams(
            use_tc_tiling_on_sc=True, needs_layout_passes=False),
    )
    def gather_kernel(idx_hbm, t3_hbm, out_hbm, idx_v0, idx_v1, g_v, o_v,
                      gsem0, gsem1, wsem0, wsem1):
        idx_v = (idx_v0, idx_v1)
        gsem = (gsem0, gsem1)
        wsem = (wsem0, wsem1)
        wid = lax.axis_index("s") * NC + lax.axis_index("c")
        c0 = wid * nc_per_w

        def stage_and_gather(i, b):
            off = pl.multiple_of((c0 + i) * _W, _W)
            pltpu.sync_copy(idx_hbm.at[pl.ds(off, _W)], idx_v[b])
            pltpu.async_copy(t3_hbm.at[idx_v[b]], g_v.at[b], gsem[b])

        def wait_gather(b):
            pltpu.make_async_copy(
                t3_hbm.at[idx_v[b]], g_v.at[b], gsem[b]).wait()

        def shuffle(b):
            g2 = g_v.at[b]
            o2 = o_v.at[b]

            def jbody(j):
                jv = jnp.full((L,), 0, jnp.int32) + j
                for k in range(D // L):
                    v = g2[j, pl.ds(k * L, L)]
                    dv = lax.iota(jnp.int32, L) + k * L
                    plsc.store_scatter(o2, [dv, jv], v)

            plsc.parallel_loop(0, _W, unroll=4)(jbody)

        def start_write(i, b):
            c = c0 + i
            h = c // chunks_per_h
            b0 = pl.multiple_of((c % chunks_per_h) * _W, _W)
            pltpu.async_copy(
                o_v.at[b, slice(None), pl.ds(0, _W)],
                out_hbm.at[h, slice(None), pl.ds(b0, _W)], wsem[b])

        def wait_write(b):
            pltpu.make_async_copy(
                o_v.at[b, slice(None), pl.ds(0, _W)],
                out_hbm.at[0, slice(None), pl.ds(0, _W)], wsem[b]).wait()

        stage_and_gather(0, 0)

        def body(go_i, carry):
            for b in range(2):
                i_var = go_i * 2 + b

                @pl.when(i_var + 1 < nc_per_w)
                def _():
                    stage_and_gather(i_var + 1, 1 - b)

                wait_gather(b)

                @pl.when(i_var >= 2)
                def _():
                    wait_write(b)

                shuffle(b)
                start_write(i_var, b)
            return carry

        lax.fori_loop(0, nc_per_w // 2, body, 0)
        wait_write(0)
        wait_write(1)

    return gather_kernel


def kernel(x, table):
    B, H = x.shape
    V, D = table.shape
    idx = x.T.reshape(B * H).astype(jnp.int32)
    tt = table.T
    ttail = lax.slice(tt, (0, V - _SLAB), (D, V))
    t3 = _make_repack(V, D)(tt, ttail)
    out_phys = _make_gather(V, D, B, H)(idx, t3)
    return jnp.transpose(out_phys, (2, 0, 1))


# speedup vs baseline: 1.4396x; 1.0361x over previous
import functools

import jax
import jax.numpy as jnp
from jax import lax
from jax.experimental import pallas as pl
from jax.experimental.pallas import tpu as pltpu
from jax.experimental.pallas import tpu_sc as plsc

_W = 256
_SLAB = 128


@functools.cache
def _make_repack(V, D):
    info = plsc.get_sparse_core_info()
    NC, NS, L = info.num_cores, info.num_subcores, info.num_lanes
    NW = NC * NS
    assert D == 64 and L == 16
    n_slabs = (V + _SLAB - 1) // _SLAB
    has_tail = V % _SLAB != 0
    mesh = plsc.VectorSubcoreMesh(core_axis_name="c", subcore_axis_name="s")

    @functools.partial(
        pl.kernel,
        mesh=mesh,
        out_type=jax.ShapeDtypeStruct((V, 2 * D), jnp.float32),
        scratch_types=[
            pltpu.VMEM((2, D, _SLAB), jnp.float32),
            pltpu.VMEM((2, _SLAB, 2 * D + 1), jnp.float32),
            pltpu.SemaphoreType.DMA,
            pltpu.SemaphoreType.DMA,
            pltpu.SemaphoreType.DMA,
            pltpu.SemaphoreType.DMA,
        ],
        compiler_params=pltpu.CompilerParams(
            use_tc_tiling_on_sc=True, needs_layout_passes=False),
    )
    def repack_kernel(tt_hbm, ttail_hbm, t3_hbm, gi_v, go_v,
                      isem0, isem1, osem0, osem1):
        isem = (isem0, isem1)
        osem = (osem0, osem1)
        wid = lax.axis_index("s") * NC + lax.axis_index("c")
        n_w = n_slabs // NW + jnp.where(wid < n_slabs % NW, 1, 0)

        def slab_of(i):
            return wid + i * NW

        def start_in(i, b):
            s = slab_of(i)
            if has_tail:
                @pl.when(s == n_slabs - 1)
                def _():
                    pltpu.async_copy(ttail_hbm, gi_v.at[b], isem[b])

                @pl.when(s < n_slabs - 1)
                def _():
                    c0 = pl.multiple_of(s * _SLAB, _SLAB)
                    pltpu.async_copy(
                        tt_hbm.at[:, pl.ds(c0, _SLAB)], gi_v.at[b], isem[b])
            else:
                c0 = pl.multiple_of(s * _SLAB, _SLAB)
                pltpu.async_copy(
                    tt_hbm.at[:, pl.ds(c0, _SLAB)], gi_v.at[b], isem[b])

        def wait_in(b):
            pltpu.make_async_copy(
                tt_hbm.at[:, pl.ds(0, _SLAB)], gi_v.at[b], isem[b]).wait()

        def transpose(b):
            gi2 = gi_v.at[b]
            go2 = go_v.at[b]

            def dbody(d):
                dv = jnp.full((L,), 0, jnp.int32) + d
                for k in range(_SLAB // L):
                    v = gi2[d, pl.ds(k * L, L)]
                    cv = lax.iota(jnp.int32, L) + k * L
                    plsc.store_scatter(go2, [cv, dv], v)

            plsc.parallel_loop(0, D, unroll=8)(dbody)

        def row0_of(i):
            s = slab_of(i)
            if has_tail:
                return jnp.where(s == n_slabs - 1, V - _SLAB, s * _SLAB)
            return s * _SLAB

        def start_out(i, b):
            r0 = pl.multiple_of(row0_of(i), 64)
            pltpu.async_copy(
                go_v.at[b, slice(None), pl.ds(0, 2 * D)],
                t3_hbm.at[pl.ds(r0, _SLAB)], osem[b])

        def wait_out(b):
            pltpu.make_async_copy(
                go_v.at[b, slice(None), pl.ds(0, 2 * D)],
                t3_hbm.at[pl.ds(0, _SLAB)], osem[b]).wait()

        start_in(0, 0)

        def body(go_i, carry):
            for b in range(2):
                i_var = go_i * 2 + b

                @pl.when(i_var + 1 < n_w)
                def _():
                    start_in(i_var + 1, 1 - b)

                @pl.when(i_var < n_w)
                def _():
                    wait_in(b)

                    @pl.when(i_var >= 2)
                    def _():
                        wait_out(b)

                    transpose(b)
                    start_out(i_var, b)
            return carry

        n_outer = (n_slabs // NW + 2) // 2
        lax.fori_loop(0, n_outer, body, 0)
        wait_out(0)
        wait_out(1)

    return repack_kernel


@functools.cache
def _make_gather(V, D, B, H):
    info = plsc.get_sparse_core_info()
    NC, NS, L = info.num_cores, info.num_subcores, info.num_lanes
    NW = NC * NS
    N = B * H
    assert D == 64 and L == 16 and B % _W == 0
    chunks_per_h = B // _W
    n_chunks = N // _W
    assert n_chunks % NW == 0
    nc_per_w = n_chunks // NW
    assert nc_per_w % 2 == 0
    mesh = plsc.VectorSubcoreMesh(core_axis_name="c", subcore_axis_name="s")

    @functools.partial(
        pl.kernel,
        mesh=mesh,
        out_type=jax.ShapeDtypeStruct((H, D, B), jnp.float32),
        scratch_types=[
            pltpu.VMEM((_W,), jnp.int32),
            pltpu.VMEM((_W,), jnp.int32),
            pltpu.VMEM((2, _W, 2 * D), jnp.float32),
            pltpu.VMEM((2, D, _W + 1), jnp.float32),
            pltpu.SemaphoreType.DMA,
            pltpu.SemaphoreType.DMA,
            pltpu.SemaphoreType.DMA,
            pltpu.SemaphoreType.DMA,
            pltpu.SemaphoreType.DMA,
            pltpu.SemaphoreType.DMA,
        ],
        compiler_params=pltpu.CompilerParams(
            use_tc_tiling_on_sc=True, needs_layout_passes=False),
    )
    def gather_kernel(idx_hbm, t3_hbm, out_hbm, idx_v0, idx_v1, g_v, o_v,
                      gsem0, gsem1, wsem0, wsem1, isem0, isem1):
        idx_v = (idx_v0, idx_v1)
        gsem = (gsem0, gsem1)
        wsem = (wsem0, wsem1)
        isem = (isem0, isem1)
        wid = lax.axis_index("s") * NC + lax.axis_index("c")
        c0 = wid * nc_per_w

        def start_idx(i, b):
            off = pl.multiple_of((c0 + i) * _W, _W)
            pltpu.async_copy(
                idx_hbm.at[pl.ds(off, _W)], idx_v[b], isem[b])

        def wait_idx(b):
            pltpu.make_async_copy(
                idx_hbm.at[pl.ds(0, _W)], idx_v[b], isem[b]).wait()

        def start_gather(b):
            pltpu.async_copy(t3_hbm.at[idx_v[b]], g_v.at[b], gsem[b])

        def wait_gather(b):
            pltpu.make_async_copy(
                t3_hbm.at[idx_v[b]], g_v.at[b], gsem[b]).wait()

        def shuffle(b):
            g2 = g_v.at[b]
            o2 = o_v.at[b]

            def jbody(j):
                jv = jnp.full((L,), 0, jnp.int32) + j
                for k in range(D // L):
                    v = g2[j, pl.ds(k * L, L)]
                    dv = lax.iota(jnp.int32, L) + k * L
                    plsc.store_scatter(o2, [dv, jv], v)

            plsc.parallel_loop(0, _W, unroll=8)(jbody)

        def start_write(i, b):
            c = c0 + i
            h = c // chunks_per_h
            b0 = pl.multiple_of((c % chunks_per_h) * _W, _W)
            pltpu.async_copy(
                o_v.at[b, slice(None), pl.ds(0, _W)],
                out_hbm.at[h, slice(None), pl.ds(b0, _W)], wsem[b])

        def wait_write(b):
            pltpu.make_async_copy(
                o_v.at[b, slice(None), pl.ds(0, _W)],
                out_hbm.at[0, slice(None), pl.ds(0, _W)], wsem[b]).wait()

        start_idx(0, 0)
        wait_idx(0)
        start_gather(0)
        start_idx(1, 1)

        def body(go_i, carry):
            for b in range(2):
                i_var = go_i * 2 + b

                @pl.when(i_var + 1 < nc_per_w)
                def _():
                    wait_idx(1 - b)
                    start_gather(1 - b)

                wait_gather(b)

                @pl.when(i_var + 2 < nc_per_w)
                def _():
                    start_idx(i_var + 2, b)

                @pl.when(i_var >= 2)
                def _():
                    wait_write(b)

                shuffle(b)
                start_write(i_var, b)
            return carry

        lax.fori_loop(0, nc_per_w // 2, body, 0)
        wait_write(0)
        wait_write(1)

    return gather_kernel


def kernel(x, table):
    B, H = x.shape
    V, D = table.shape
    idx = x.T.reshape(B * H).astype(jnp.int32)
    tt = table.T
    ttail = lax.slice(tt, (0, V - _SLAB), (D, V))
    t3 = _make_repack(V, D)(tt, ttail)
    out_phys = _make_gather(V, D, B, H)(idx, t3)
    return jnp.transpose(out_phys, (2, 0, 1))


# speedup vs baseline: 1.4403x; 1.0005x over previous
import functools

import jax
import jax.numpy as jnp
from jax import lax
from jax.experimental import pallas as pl
from jax.experimental.pallas import tpu as pltpu
from jax.experimental.pallas import tpu_sc as plsc

_W = 256
_SLAB = 128


@functools.cache
def _make_repack(V, D):
    info = plsc.get_sparse_core_info()
    NC, NS, L = info.num_cores, info.num_subcores, info.num_lanes
    NW = NC * NS
    assert D == 64 and L == 16
    n_slabs = (V + _SLAB - 1) // _SLAB
    has_tail = V % _SLAB != 0
    mesh = plsc.VectorSubcoreMesh(core_axis_name="c", subcore_axis_name="s")

    @functools.partial(
        pl.kernel,
        mesh=mesh,
        out_type=jax.ShapeDtypeStruct((V, 2 * D), jnp.float32),
        scratch_types=[
            pltpu.VMEM((2, D, _SLAB), jnp.float32),
            pltpu.VMEM((2, _SLAB, 136), jnp.float32),
            pltpu.SemaphoreType.DMA,
            pltpu.SemaphoreType.DMA,
            pltpu.SemaphoreType.DMA,
            pltpu.SemaphoreType.DMA,
        ],
        compiler_params=pltpu.CompilerParams(
            use_tc_tiling_on_sc=True, needs_layout_passes=False),
    )
    def repack_kernel(tt_hbm, ttail_hbm, t3_hbm, gi_v, go_v,
                      isem0, isem1, osem0, osem1):
        isem = (isem0, isem1)
        osem = (osem0, osem1)
        wid = lax.axis_index("s") * NC + lax.axis_index("c")
        n_w = n_slabs // NW + jnp.where(wid < n_slabs % NW, 1, 0)

        def slab_of(i):
            return wid + i * NW

        def start_in(i, b):
            s = slab_of(i)
            if has_tail:
                @pl.when(s == n_slabs - 1)
                def _():
                    pltpu.async_copy(ttail_hbm, gi_v.at[b], isem[b])

                @pl.when(s < n_slabs - 1)
                def _():
                    c0 = pl.multiple_of(s * _SLAB, _SLAB)
                    pltpu.async_copy(
                        tt_hbm.at[:, pl.ds(c0, _SLAB)], gi_v.at[b], isem[b])
            else:
                c0 = pl.multiple_of(s * _SLAB, _SLAB)
                pltpu.async_copy(
                    tt_hbm.at[:, pl.ds(c0, _SLAB)], gi_v.at[b], isem[b])

        def wait_in(b):
            pltpu.make_async_copy(
                tt_hbm.at[:, pl.ds(0, _SLAB)], gi_v.at[b], isem[b]).wait()

        def transpose(b):
            gi2 = gi_v.at[b]
            go2 = go_v.at[b]

            def dbody(d):
                dv = jnp.full((L,), 0, jnp.int32) + d
                for k in range(_SLAB // L):
                    v = gi2[d, pl.ds(k * L, L)]
                    cv = lax.iota(jnp.int32, L) + k * L
                    plsc.store_scatter(go2, [cv, dv], v)

            plsc.parallel_loop(0, D, unroll=8)(dbody)

        def row0_of(i):
            s = slab_of(i)
            if has_tail:
                return jnp.where(s == n_slabs - 1, V - _SLAB, s * _SLAB)
            return s * _SLAB

        def start_out(i, b):
            r0 = pl.multiple_of(row0_of(i), 64)
            pltpu.async_copy(
                go_v.at[b, slice(None), pl.ds(0, 2 * D)],
                t3_hbm.at[pl.ds(r0, _SLAB)], osem[b])

        def wait_out(b):
            pltpu.make_async_copy(
                go_v.at[b, slice(None), pl.ds(0, 2 * D)],
                t3_hbm.at[pl.ds(0, _SLAB)], osem[b]).wait()

        start_in(0, 0)

        def body(go_i, carry):
            for b in range(2):
                i_var = go_i * 2 + b

                @pl.when(i_var + 1 < n_w)
                def _():
                    start_in(i_var + 1, 1 - b)

                @pl.when(i_var < n_w)
                def _():
                    wait_in(b)

                    @pl.when(i_var >= 2)
                    def _():
                        wait_out(b)

                    transpose(b)
                    start_out(i_var, b)
            return carry

        n_outer = (n_slabs // NW + 2) // 2
        lax.fori_loop(0, n_outer, body, 0)
        wait_out(0)
        wait_out(1)

    return repack_kernel


@functools.cache
def _make_gather(V, D, B, H):
    info = plsc.get_sparse_core_info()
    NC, NS, L = info.num_cores, info.num_subcores, info.num_lanes
    NW = NC * NS
    N = B * H
    assert D == 64 and L == 16 and B % _W == 0
    chunks_per_h = B // _W
    n_chunks = N // _W
    assert n_chunks % NW == 0
    nc_per_w = n_chunks // NW
    assert nc_per_w % 2 == 0
    mesh = plsc.VectorSubcoreMesh(core_axis_name="c", subcore_axis_name="s")

    @functools.partial(
        pl.kernel,
        mesh=mesh,
        out_type=jax.ShapeDtypeStruct((H, D, B), jnp.float32),
        scratch_types=[
            pltpu.VMEM((_W,), jnp.int32),
            pltpu.VMEM((_W,), jnp.int32),
            pltpu.VMEM((2, _W, 2 * D), jnp.float32),
            pltpu.VMEM((2, D, _W + 8), jnp.float32),
            pltpu.SemaphoreType.DMA,
            pltpu.SemaphoreType.DMA,
            pltpu.SemaphoreType.DMA,
            pltpu.SemaphoreType.DMA,
            pltpu.SemaphoreType.DMA,
            pltpu.SemaphoreType.DMA,
        ],
        compiler_params=pltpu.CompilerParams(
            use_tc_tiling_on_sc=True, needs_layout_passes=False),
    )
    def gather_kernel(idx_hbm, t3_hbm, out_hbm, idx_v0, idx_v1, g_v, o_v,
                      gsem0, gsem1, wsem0, wsem1, isem0, isem1):
        idx_v = (idx_v0, idx_v1)
        gsem = (gsem0, gsem1)
        wsem = (wsem0, wsem1)
        isem = (isem0, isem1)
        wid = lax.axis_index("s") * NC + lax.axis_index("c")
        c0 = wid * nc_per_w

        def start_idx(i, b):
            off = pl.multiple_of((c0 + i) * _W, _W)
            pltpu.async_copy(
                idx_hbm.at[pl.ds(off, _W)], idx_v[b], isem[b])

        def wait_idx(b):
            pltpu.make_async_copy(
                idx_hbm.at[pl.ds(0, _W)], idx_v[b], isem[b]).wait()

        def start_gather(b):
            pltpu.async_copy(t3_hbm.at[idx_v[b]], g_v.at[b], gsem[b])

        def wait_gather(b):
            pltpu.make_async_copy(
                t3_hbm.at[idx_v[b]], g_v.at[b], gsem[b]).wait()

        def shuffle(b):
            g2 = g_v.at[b]
            o2 = o_v.at[b]

            def jbody(j):
                jv = jnp.full((L,), 0, jnp.int32) + j
                for k in range(D // L):
                    v = g2[j, pl.ds(k * L, L)]
                    dv = lax.iota(jnp.int32, L) + k * L
                    plsc.store_scatter(o2, [dv, jv], v)

            plsc.parallel_loop(0, _W, unroll=8)(jbody)

        def start_write(i, b):
            c = c0 + i
            h = c // chunks_per_h
            b0 = pl.multiple_of((c % chunks_per_h) * _W, _W)
            pltpu.async_copy(
                o_v.at[b, slice(None), pl.ds(0, _W)],
                out_hbm.at[h, slice(None), pl.ds(b0, _W)], wsem[b])

        def wait_write(b):
            pltpu.make_async_copy(
                o_v.at[b, slice(None), pl.ds(0, _W)],
                out_hbm.at[0, slice(None), pl.ds(0, _W)], wsem[b]).wait()

        start_idx(0, 0)
        wait_idx(0)
        start_gather(0)
        start_idx(1, 1)

        def body(go_i, carry):
            for b in range(2):
                i_var = go_i * 2 + b

                @pl.when(i_var + 1 < nc_per_w)
                def _():
                    wait_idx(1 - b)
                    start_gather(1 - b)

                wait_gather(b)

                @pl.when(i_var + 2 < nc_per_w)
                def _():
                    start_idx(i_var + 2, b)

                @pl.when(i_var >= 2)
                def _():
                    wait_write(b)

                shuffle(b)
                start_write(i_var, b)
            return carry

        lax.fori_loop(0, nc_per_w // 2, body, 0)
        wait_write(0)
        wait_write(1)

    return gather_kernel


def kernel(x, table):
    B, H = x.shape
    V, D = table.shape
    idx = x.T.reshape(B * H).astype(jnp.int32)
    tt = table.T
    ttail = lax.slice(tt, (0, V - _SLAB), (D, V))
    t3 = _make_repack(V, D)(tt, ttail)
    out_phys = _make_gather(V, D, B, H)(idx, t3)
    return jnp.transpose(out_phys, (2, 0, 1))


# speedup vs baseline: 1.8427x; 1.2794x over previous
import functools

import jax
import jax.numpy as jnp
from jax import lax
from jax.experimental import pallas as pl
from jax.experimental.pallas import tpu as pltpu
from jax.experimental.pallas import tpu_sc as plsc

_NBUF = 2
_CH = 640


@functools.cache
def _make_gather(V, D, B):
    info = plsc.get_sparse_core_info()
    NC, NS = info.num_cores, info.num_subcores
    NW = NC * NS
    assert B % NW == 0
    b_per_w = B // NW
    assert b_per_w % (_CH * _NBUF) == 0
    n_ch = b_per_w // _CH
    mesh = plsc.VectorSubcoreMesh(core_axis_name="c", subcore_axis_name="s")

    @functools.partial(
        pl.kernel,
        mesh=mesh,
        out_type=jax.ShapeDtypeStruct((B, D), jnp.float32),
        scratch_types=[
            pltpu.VMEM((b_per_w,), jnp.int32),
            pltpu.VMEM((_NBUF, _CH, D), jnp.float32),
            pltpu.SemaphoreType.DMA,
            pltpu.SemaphoreType.DMA,
            pltpu.SemaphoreType.DMA,
            pltpu.SemaphoreType.DMA,
        ],
        compiler_params=pltpu.CompilerParams(use_tc_tiling_on_sc=False),
    )
    def gather_kernel(idx_hbm, table_hbm, out_hbm, idx_v, rows_v,
                      gsem0, gsem1, wsem0, wsem1):
        gsem = (gsem0, gsem1)
        wsem = (wsem0, wsem1)
        wid = lax.axis_index("s") * NC + lax.axis_index("c")
        base = wid * b_per_w
        pltpu.sync_copy(idx_hbm.at[pl.ds(base, b_per_w)], idx_v)

        def start_gather(i, b):
            off = pl.multiple_of(i * _CH, _CH)
            pltpu.async_copy(
                table_hbm.at[idx_v.at[pl.ds(off, _CH)]], rows_v.at[b], gsem[b])

        def wait_gather(b):
            pltpu.make_async_copy(
                table_hbm.at[idx_v.at[pl.ds(0, _CH)]], rows_v.at[b],
                gsem[b]).wait()

        def start_write(i, b):
            off = pl.multiple_of(base + i * _CH, _CH)
            pltpu.async_copy(rows_v.at[b], out_hbm.at[pl.ds(off, _CH)], wsem[b])

        def wait_write(b):
            pltpu.make_async_copy(
                rows_v.at[b], out_hbm.at[pl.ds(0, _CH)], wsem[b]).wait()

        start_gather(0, 0)
        start_gather(1, 1)
        wait_gather(0)
        start_write(0, 0)
        wait_gather(1)
        start_write(1, 1)

        def body(g, carry):
            for b in range(_NBUF):
                i = g * _NBUF + b
                wait_write(b)
                start_gather(i, b)
                wait_gather(b)
                start_write(i, b)
            return carry

        lax.fori_loop(1, n_ch // _NBUF, body, 0)
        wait_write(0)
        wait_write(1)

    return gather_kernel


def kernel(x, table):
    B, H = x.shape
    V, D = table.shape
    idx = x.reshape(B * H).astype(jnp.int32)
    out = _make_gather(V, D, B * H)(idx, table)
    return out.reshape(B, H, D)
